# Initial kernel scaffold; baseline (speedup 1.0000x reference)
import jax
import jax.numpy as jnp
from jax.experimental import pallas as pl


def kernel(x, W_proj, W2, b2):
    raise NotImplementedError("write your pallas kernel here")


# speedup vs baseline: 22.6295x; 22.6295x over previous
import functools

import jax
import jax.numpy as jnp
from jax import lax
from jax.experimental import pallas as pl
from jax.experimental.pallas import tpu as pltpu

K_WINNERS = 32


def _fused_body(x_ref, wp_ref, w2_ref, b2_ref, out_ref, y_scr):
    y = lax.dot_general(
        x_ref[...], wp_ref[...],
        (((1,), (1,)), ((), ())),
        preferred_element_type=jnp.float32,
    )
    y_scr[...] = y

    rowmax = jnp.max(y, axis=1, keepdims=True)
    hi0 = lax.bitcast_convert_type(rowmax, jnp.int32) + 1
    lo0 = jnp.zeros_like(hi0)

    def body(_, carry):
        lo, hi = carry
        mid = lo + (hi - lo) // 2
        midf = lax.bitcast_convert_type(mid, jnp.float32)
        cnt = jnp.sum(
            (y_scr[...] >= midf).astype(jnp.int32), axis=1, keepdims=True
        )
        pred = cnt >= K_WINNERS
        lo = jnp.where(pred, mid, lo)
        hi = jnp.where(pred, hi, mid)
        return lo, hi

    lo, _ = lax.fori_loop(0, 31, body, (lo0, hi0))
    thresh = lax.bitcast_convert_type(lo, jnp.float32)

    yv = y_scr[...]
    sparse = jnp.where(yv >= thresh, yv, 0.0)
    w2c = jnp.maximum(w2_ref[...], 0.0)
    out = lax.dot_general(
        sparse, w2c,
        (((1,), (1,)), ((), ())),
        preferred_element_type=jnp.float32,
    )
    out_ref[...] = out + b2_ref[...]


@functools.partial(jax.jit, static_argnames=("bt",))
def _run(x, W_proj, W2, b2, bt=128):
    B, F = x.shape
    N = W_proj.shape[0]
    C = W2.shape[0]
    grid = (B // bt,)
    return pl.pallas_call(
        _fused_body,
        grid=grid,
        in_specs=[
            pl.BlockSpec((bt, F), lambda i: (i, 0)),
            pl.BlockSpec((N, F), lambda i: (0, 0)),
            pl.BlockSpec((C, N), lambda i: (0, 0)),
            pl.BlockSpec((1, C), lambda i: (0, 0)),
        ],
        out_specs=pl.BlockSpec((bt, C), lambda i: (i, 0)),
        out_shape=jax.ShapeDtypeStruct((B, C), jnp.float32),
        scratch_shapes=[pltpu.VMEM((bt, N), jnp.float32)],
        compiler_params=pltpu.CompilerParams(
            vmem_limit_bytes=63 * 1024 * 1024,
        ),
    )(x, W_proj, W2, b2.reshape(1, C))


def kernel(x, W_proj, W2, b2):
    return _run(x, W_proj, W2, b2)


# speedup vs baseline: 25.1737x; 1.1124x over previous
import functools

import jax
import jax.numpy as jnp
from jax import lax
from jax.experimental import pallas as pl
from jax.experimental.pallas import tpu as pltpu

K_WINNERS = 32


def _fused_body(x_ref, wp_ref, w2_ref, b2_ref, out_ref, y_scr):
    y = lax.dot_general(
        x_ref[...], wp_ref[...],
        (((1,), (1,)), ((), ())),
        preferred_element_type=jnp.float32,
    )
    y_scr[...] = y

    rowmax = jnp.max(y, axis=1, keepdims=True)
    hi_row = lax.bitcast_convert_type(rowmax, jnp.int32) + 1

    SUB = 1280
    ysub = y_scr[:, :SUB]

    def body_a(_, carry):
        lo, hi = carry
        mid = lo + (hi - lo) // 2
        midf = lax.bitcast_convert_type(mid, jnp.float32)
        cnt = jnp.sum(
            (ysub >= midf).astype(jnp.float32), axis=1, keepdims=True
        )
        pred = cnt >= K_WINNERS
        return jnp.where(pred, mid, lo), jnp.where(pred, hi, mid)

    lo_a, _ = lax.fori_loop(
        0, 31, body_a, (jnp.zeros_like(hi_row), hi_row)
    )

    ones_n = jnp.ones((1, y.shape[1]), jnp.float32)

    def cond_b(carry):
        lo, hi = carry
        return jnp.max(hi - lo) > 1

    def body_b(carry):
        lo, hi = carry
        mid = lo + (hi - lo) // 2
        midf = lax.bitcast_convert_type(mid, jnp.float32)
        maskf = (y_scr[...] >= midf).astype(jnp.float32)
        cnt = lax.dot_general(
            maskf, ones_n,
            (((1,), (1,)), ((), ())),
            preferred_element_type=jnp.float32,
        )
        pred = cnt >= K_WINNERS
        return jnp.where(pred, mid, lo), jnp.where(pred, hi, mid)

    lo, _ = lax.while_loop(cond_b, body_b, (lo_a, hi_row))
    thresh = lax.bitcast_convert_type(lo, jnp.float32)

    yv = y_scr[...]
    sparse = jnp.where(yv >= thresh, yv, 0.0)
    w2c = jnp.maximum(w2_ref[...], 0.0)
    out = lax.dot_general(
        sparse, w2c,
        (((1,), (1,)), ((), ())),
        preferred_element_type=jnp.float32,
    )
    out_ref[...] = out + b2_ref[...]


@functools.partial(jax.jit, static_argnames=("bt",))
def _run(x, W_proj, W2, b2, bt=128):
    B, F = x.shape
    N = W_proj.shape[0]
    C = W2.shape[0]
    grid = (B // bt,)
    return pl.pallas_call(
        _fused_body,
        grid=grid,
        in_specs=[
            pl.BlockSpec((bt, F), lambda i: (i, 0)),
            pl.BlockSpec((N, F), lambda i: (0, 0)),
            pl.BlockSpec((C, N), lambda i: (0, 0)),
            pl.BlockSpec((1, C), lambda i: (0, 0)),
        ],
        out_specs=pl.BlockSpec((bt, C), lambda i: (i, 0)),
        out_shape=jax.ShapeDtypeStruct((B, C), jnp.float32),
        scratch_shapes=[pltpu.VMEM((bt, N), jnp.float32)],
        compiler_params=pltpu.CompilerParams(
            vmem_limit_bytes=63 * 1024 * 1024,
        ),
    )(x, W_proj, W2, b2.reshape(1, C))


def kernel(x, W_proj, W2, b2):
    return _run(x, W_proj, W2, b2)
